# baseline (speedup 1.0000x reference)
import functools

import jax
import jax.numpy as jnp
from jax import lax
from jax.experimental import pallas as pl
from jax.experimental.pallas import tpu as pltpu
from jax.experimental.pallas import tpu_sc as plsc

EMB_D = 32
LANE = 128
NBT = 8
BLK = LANE * NBT
NDT = EMB_D // 8


def _sc_embed(table, idx_t):
    H, nrow, lane = idx_t.shape
    assert lane == LANE and nrow % NBT == 0
    n_blk = nrow // NBT
    n_units = H * n_blk
    info = plsc.get_sparse_core_info()
    nc, ns = info.num_cores, info.num_subcores
    nw = nc * ns
    units_per_w = n_units // nw
    assert units_per_w * nw == n_units

    mesh = plsc.VectorSubcoreMesh(core_axis_name="c", subcore_axis_name="s")

    @functools.partial(
        pl.kernel,
        mesh=mesh,
        out_type=jax.ShapeDtypeStruct((H, NDT, nrow, 8, LANE), jnp.float32),
        scratch_types=[
            pltpu.VMEM((NBT, LANE), jnp.int32),
            pltpu.VMEM((BLK, EMB_D), jnp.float32),
            pltpu.VMEM((NDT, NBT, 8, LANE), jnp.float32),
            pltpu.SemaphoreType.DMA,
            pltpu.SemaphoreType.DMA,
        ],
        compiler_params=pltpu.CompilerParams(
            use_tc_tiling_on_sc=False, needs_layout_passes=False
        ),
    )
    def body(table_hbm, idx_hbm, out_hbm, idx_v, rows_v, trans_v, gsem, osem):
        wid = lax.axis_index("s") * nc + lax.axis_index("c")
        u0 = wid * units_per_w
        iota = lax.iota(jnp.int32, 16)

        def unit_body(i, carry):
            u = u0 + i
            h = u // n_blk
            blk = u % n_blk
            r = blk * NBT

            pltpu.sync_copy(idx_hbm.at[h, pl.ds(r, NBT)], idx_v)
            for j in range(NBT):
                pltpu.async_copy(
                    table_hbm.at[idx_v.at[j]],
                    rows_v.at[pl.ds(j * LANE, LANE)],
                    gsem,
                )
            pltpu.make_async_copy(
                out_hbm.at[h, 0, pl.ds(0, NBT * NDT)], rows_v, gsem
            ).wait()

            def bt_body(bt, c2):
                base = bt * LANE
                for dt in range(NDT):
                    def dr_body(dr, c3):
                        d1 = jnp.full((16,), dt * 8, jnp.int32) + dr
                        for j0 in range(8):
                            v = plsc.load_gather(
                                rows_v, [iota + (base + j0 * 16), d1]
                            )
                            trans_v[dt, bt, dr, pl.ds(j0 * 16, 16)] = v
                        return c3
                    lax.fori_loop(0, 8, dr_body, 0)
                return c2
            lax.fori_loop(0, NBT, bt_body, 0)

            for dt in range(NDT):
                pltpu.async_copy(
                    trans_v.at[dt], out_hbm.at[h, dt, pl.ds(r, NBT)], osem
                )
            for dt in range(NDT):
                pltpu.make_async_copy(
                    trans_v.at[dt], out_hbm.at[h, dt, pl.ds(r, NBT)], osem
                ).wait()
            return carry

        lax.fori_loop(0, units_per_w, unit_body, 0)

    return body(table, idx_t)


def kernel(indices, table):
    nb, H = indices.shape
    assert nb % BLK == 0 and table.shape[1] == EMB_D
    idx_t = jnp.transpose(indices).reshape(H, nb // LANE, LANE)
    x5 = _sc_embed(table, idx_t.astype(jnp.int32))
    return x5.transpose(2, 4, 0, 1, 3).reshape(nb, H, EMB_D)


# speedup vs baseline: 1.1071x; 1.1071x over previous
import functools

import jax
import jax.numpy as jnp
from jax import lax
from jax.experimental import pallas as pl
from jax.experimental.pallas import tpu as pltpu
from jax.experimental.pallas import tpu_sc as plsc

EMB_D = 32
LANE = 128
NBT = 8
BLK = LANE * NBT
NDT = EMB_D // 8


def _sc_embed(table, idx_t):
    H, nrow, lane = idx_t.shape
    assert lane == LANE and nrow % NBT == 0
    n_blk = nrow // NBT
    n_units = H * n_blk
    info = plsc.get_sparse_core_info()
    nc, ns = info.num_cores, info.num_subcores
    nw = nc * ns
    units_per_w = n_units // nw
    assert units_per_w * nw == n_units and units_per_w % 2 == 0

    mesh = plsc.VectorSubcoreMesh(core_axis_name="c", subcore_axis_name="s")

    @functools.partial(
        pl.kernel,
        mesh=mesh,
        out_type=jax.ShapeDtypeStruct((H, NDT, nrow, 8, LANE), jnp.float32),
        scratch_types=[
            pltpu.VMEM((NBT, LANE), jnp.int32),
            pltpu.VMEM((NBT, LANE), jnp.int32),
            pltpu.VMEM((BLK, EMB_D), jnp.float32),
            pltpu.VMEM((BLK, EMB_D), jnp.float32),
            pltpu.VMEM((NDT, NBT, 8, LANE), jnp.float32),
            pltpu.SemaphoreType.DMA,
            pltpu.SemaphoreType.DMA,
            pltpu.SemaphoreType.DMA,
        ],
        compiler_params=pltpu.CompilerParams(
            use_tc_tiling_on_sc=False, needs_layout_passes=False
        ),
    )
    def body(table_hbm, idx_hbm, out_hbm, idx_v0, idx_v1, rows_v0, rows_v1,
             trans_v, gs0, gs1, osem):
        wid = lax.axis_index("s") * nc + lax.axis_index("c")
        u0 = wid * units_per_w
        idx_vs = (idx_v0, idx_v1)
        rows_vs = (rows_v0, rows_v1)
        gsems = (gs0, gs1)
        iotas = [lax.iota(jnp.int32, 16) + (16 * j0) for j0 in range(8)]

        def fire_unit(u, slot):
            h = u // n_blk
            r = (u % n_blk) * NBT
            pltpu.sync_copy(idx_hbm.at[h, pl.ds(r, NBT)], idx_vs[slot])
            for j in range(NBT):
                pltpu.async_copy(
                    table_hbm.at[idx_vs[slot].at[j]],
                    rows_vs[slot].at[pl.ds(j * LANE, LANE)],
                    gsems[slot],
                )

        def wait_unit(slot):
            pltpu.make_async_copy(
                out_hbm.at[0, 0, pl.ds(0, NBT * NDT)], rows_vs[slot], gsems[slot]
            ).wait()

        def transpose_block(slot):
            rows = rows_vs[slot]

            def bt_body(bt, c):
                base = bt * LANE
                idx0s = [iotas[j0] + base for j0 in range(8)]
                for d in range(EMB_D):
                    ds = jnp.full((16,), d, jnp.int32)
                    dt, dr = d // 8, d % 8
                    for j0 in range(8):
                        v = plsc.load_gather(rows, [idx0s[j0], ds])
                        trans_v[dt, bt, dr, pl.ds(j0 * 16, 16)] = v
                return c

            lax.fori_loop(0, NBT, bt_body, 0)

        def store_unit(u):
            h = u // n_blk
            r = (u % n_blk) * NBT
            for dt in range(NDT):
                pltpu.async_copy(
                    trans_v.at[dt], out_hbm.at[h, dt, pl.ds(r, NBT)], osem
                )

        def wait_store(u):
            h = u // n_blk
            r = (u % n_blk) * NBT
            for dt in range(NDT):
                pltpu.make_async_copy(
                    trans_v.at[dt], out_hbm.at[h, dt, pl.ds(r, NBT)], osem
                ).wait()

        fire_unit(u0, 0)

        def outer(g, carry):
            for b in range(2):
                i = g * 2 + b
                u = u0 + i
                slot = b
                wait_unit(slot)

                @pl.when(i + 1 < units_per_w)
                def _():
                    fire_unit(u + 1, 1 - slot)

                @pl.when(i >= 1)
                def _():
                    wait_store(u - 1)

                transpose_block(slot)
                store_unit(u)
            return carry

        lax.fori_loop(0, units_per_w // 2, outer, 0)
        wait_store(u0 + units_per_w - 1)

    return body(table, idx_t)


def kernel(indices, table):
    nb, H = indices.shape
    assert nb % BLK == 0 and table.shape[1] == EMB_D
    idx_t = jnp.transpose(indices).reshape(H, nb // LANE, LANE)
    x5 = _sc_embed(table, idx_t.astype(jnp.int32))
    return x5.transpose(2, 4, 0, 1, 3).reshape(nb, H, EMB_D)


# speedup vs baseline: 1.3647x; 1.2327x over previous
import functools

import jax
import jax.numpy as jnp
from jax import lax
from jax.experimental import pallas as pl
from jax.experimental.pallas import tpu as pltpu
from jax.experimental.pallas import tpu_sc as plsc

EMB_D = 32
LANE = 128
NBT = 8
BLK = LANE * NBT
NDT = EMB_D // 8


def _sc_embed(table, idx_t):
    H, nrow, lane = idx_t.shape
    assert lane == LANE and nrow % NBT == 0
    n_blk = nrow // NBT
    n_units = H * n_blk
    info = plsc.get_sparse_core_info()
    nc, ns = info.num_cores, info.num_subcores
    nw = nc * ns
    units_per_w = n_units // nw
    assert units_per_w * nw == n_units and units_per_w % 2 == 0

    mesh = plsc.VectorSubcoreMesh(core_axis_name="c", subcore_axis_name="s")

    @functools.partial(
        pl.kernel,
        mesh=mesh,
        out_type=jax.ShapeDtypeStruct((H, NDT, nrow, 8, LANE), jnp.float32),
        scratch_types=[
            pltpu.VMEM((NBT, LANE), jnp.int32),
            pltpu.VMEM((NBT, LANE), jnp.int32),
            pltpu.VMEM((BLK, EMB_D), jnp.float32),
            pltpu.VMEM((BLK, EMB_D), jnp.float32),
            pltpu.VMEM((NDT, NBT, 8, LANE), jnp.float32),
            pltpu.SemaphoreType.DMA,
            pltpu.SemaphoreType.DMA,
            pltpu.SemaphoreType.DMA,
        ],
        compiler_params=pltpu.CompilerParams(
            use_tc_tiling_on_sc=False, needs_layout_passes=False
        ),
    )
    def body(table_hbm, idx_hbm, out_hbm, idx_v0, idx_v1, rows_v0, rows_v1,
             trans_v, gs0, gs1, osem):
        wid = lax.axis_index("s") * nc + lax.axis_index("c")
        u0 = wid * units_per_w
        idx_vs = (idx_v0, idx_v1)
        rows_vs = (rows_v0, rows_v1)
        gsems = (gs0, gs1)
        iotas = [lax.iota(jnp.int32, 16) + (16 * j0) for j0 in range(8)]

        def fire_unit(u, slot):
            h = u // n_blk
            r = (u % n_blk) * NBT
            pltpu.sync_copy(idx_hbm.at[h, pl.ds(r, NBT)], idx_vs[slot])
            for j in range(NBT):
                pltpu.async_copy(
                    table_hbm.at[idx_vs[slot].at[j]],
                    rows_vs[slot].at[pl.ds(j * LANE, LANE)],
                    gsems[slot],
                )

        def wait_unit(slot):
            pltpu.make_async_copy(
                out_hbm.at[0, 0, pl.ds(0, NBT * NDT)], rows_vs[slot], gsems[slot]
            ).wait()

        def transpose_block(slot):
            rows = rows_vs[slot]

            def bt_body(bt, c):
                base = bt * LANE
                idx0s = [iotas[j0] + base for j0 in range(8)]
                for d in range(EMB_D):
                    ds = jnp.full((16,), d, jnp.int32)
                    dt, dr = d // 8, d % 8
                    vs = [
                        plsc.load_gather(rows, [idx0s[j0], ds])
                        for j0 in range(8)
                    ]
                    for j0 in range(8):
                        trans_v[dt, bt, dr, pl.ds(j0 * 16, 16)] = vs[j0]
                return c

            lax.fori_loop(0, NBT, bt_body, 0)

        def store_unit(u):
            h = u // n_blk
            r = (u % n_blk) * NBT
            for dt in range(NDT):
                pltpu.async_copy(
                    trans_v.at[dt], out_hbm.at[h, dt, pl.ds(r, NBT)], osem
                )

        def wait_store(u):
            h = u // n_blk
            r = (u % n_blk) * NBT
            for dt in range(NDT):
                pltpu.make_async_copy(
                    trans_v.at[dt], out_hbm.at[h, dt, pl.ds(r, NBT)], osem
                ).wait()

        fire_unit(u0, 0)

        def outer(g, carry):
            for b in range(2):
                i = g * 2 + b
                u = u0 + i
                slot = b
                wait_unit(slot)

                @pl.when(i + 1 < units_per_w)
                def _():
                    fire_unit(u + 1, 1 - slot)

                @pl.when(i >= 1)
                def _():
                    wait_store(u - 1)

                transpose_block(slot)
                store_unit(u)
            return carry

        lax.fori_loop(0, units_per_w // 2, outer, 0)
        wait_store(u0 + units_per_w - 1)

    return body(table, idx_t)


def kernel(indices, table):
    nb, H = indices.shape
    assert nb % BLK == 0 and table.shape[1] == EMB_D
    idx_t = jnp.transpose(indices).reshape(H, nb // LANE, LANE)
    x5 = _sc_embed(table, idx_t.astype(jnp.int32))
    return x5.transpose(2, 4, 0, 1, 3).reshape(nb, H, EMB_D)


# speedup vs baseline: 5.6163x; 4.1153x over previous
import functools

import jax
import jax.numpy as jnp
from jax import lax
from jax.experimental import pallas as pl
from jax.experimental.pallas import tpu as pltpu
from jax.experimental.pallas import tpu_sc as plsc

EMB_D = 32
LANE = 128
NBT = 8
BLK = LANE * NBT
NDT = EMB_D // 8


def _sc_embed(table, idx_t):
    H, nrow, lane = idx_t.shape
    assert lane == LANE and nrow % NBT == 0
    n_blk = nrow // NBT
    n_units = H * n_blk
    info = plsc.get_sparse_core_info()
    nc, ns = info.num_cores, info.num_subcores
    nw = nc * ns
    units_per_w = n_units // nw
    assert units_per_w * nw == n_units and units_per_w % 2 == 0

    mesh = plsc.VectorSubcoreMesh(core_axis_name="c", subcore_axis_name="s")

    @functools.partial(
        pl.kernel,
        mesh=mesh,
        out_type=jax.ShapeDtypeStruct((H, NDT, nrow * 8, LANE), jnp.float32),
        scratch_types=[
            pltpu.VMEM((NBT, LANE), jnp.int32),
            pltpu.VMEM((NBT, LANE), jnp.int32),
            pltpu.VMEM((BLK, EMB_D), jnp.float32),
            pltpu.VMEM((BLK, EMB_D), jnp.float32),
            pltpu.VMEM((NDT * NBT * 8, LANE), jnp.float32),
            pltpu.SemaphoreType.DMA,
            pltpu.SemaphoreType.DMA,
            pltpu.SemaphoreType.DMA,
        ],
        compiler_params=pltpu.CompilerParams(
            use_tc_tiling_on_sc=False, needs_layout_passes=False
        ),
    )
    def body(table_hbm, idx_hbm, out_hbm, idx_v0, idx_v1, rows_v0, rows_v1,
             trans_v, gs0, gs1, osem):
        wid = lax.axis_index("s") * nc + lax.axis_index("c")
        u0 = wid * units_per_w
        idx_vs = (idx_v0, idx_v1)
        rows_vs = (rows_v0, rows_v1)
        gsems = (gs0, gs1)
        iota = lax.iota(jnp.int32, 16)
        ms = [(iota + k) % 16 for k in range(16)]
        rowc = [(m // 8) * (NBT * 8) + (m % 8) for m in ms]

        def fire_unit(u, slot):
            h = u // n_blk
            r = (u % n_blk) * NBT
            pltpu.sync_copy(idx_hbm.at[h, pl.ds(r, NBT)], idx_vs[slot])
            for j in range(NBT):
                pltpu.async_copy(
                    table_hbm.at[idx_vs[slot].at[j]],
                    rows_vs[slot].at[pl.ds(j * LANE, LANE)],
                    gsems[slot],
                )

        def wait_unit(slot):
            pltpu.make_async_copy(
                table_hbm.at[pl.ds(0, BLK)], rows_vs[slot], gsems[slot]
            ).wait()

        def transpose_block(slot):
            rows = rows_vs[slot]

            def bt_body(bt, c):
                base = bt * LANE
                for g16 in range(8):
                    toks = iota + (base + g16 * 16)
                    col = iota + (g16 * 16)
                    for d0 in (0, 16):
                        srow = (d0 // 8) * (NBT * 8) + bt * 8
                        vs = []
                        for k in range(16):
                            v = plsc.load_gather(rows, [toks, ms[k] + d0])
                            vs.append(v)
                        for k in range(16):
                            plsc.store_scatter(
                                trans_v, [rowc[k] + srow, col], vs[k]
                            )
                return c

            lax.fori_loop(0, NBT, bt_body, 0)

        def store_unit(u):
            h = u // n_blk
            r = (u % n_blk) * NBT * 8
            for dt in range(NDT):
                pltpu.async_copy(
                    trans_v.at[pl.ds(dt * NBT * 8, NBT * 8)],
                    out_hbm.at[h, dt, pl.ds(r, NBT * 8)],
                    osem,
                )

        def wait_store(u):
            h = u // n_blk
            r = (u % n_blk) * NBT * 8
            for dt in range(NDT):
                pltpu.make_async_copy(
                    trans_v.at[pl.ds(dt * NBT * 8, NBT * 8)],
                    out_hbm.at[h, dt, pl.ds(r, NBT * 8)],
                    osem,
                ).wait()

        fire_unit(u0, 0)

        def outer(g, carry):
            for b in range(2):
                i = g * 2 + b
                u = u0 + i
                slot = b
                wait_unit(slot)

                @pl.when(i + 1 < units_per_w)
                def _():
                    fire_unit(u + 1, 1 - slot)

                @pl.when(i >= 1)
                def _():
                    wait_store(u - 1)

                transpose_block(slot)
                store_unit(u)
            return carry

        lax.fori_loop(0, units_per_w // 2, outer, 0)
        wait_store(u0 + units_per_w - 1)

    return body(table, idx_t)


def kernel(indices, table):
    nb, H = indices.shape
    assert nb % BLK == 0 and table.shape[1] == EMB_D
    idx_t = jnp.transpose(indices).reshape(H, nb // LANE, LANE)
    x4 = _sc_embed(table, idx_t.astype(jnp.int32))
    x5 = x4.reshape(H, NDT, nb // LANE, 8, LANE)
    return x5.transpose(2, 4, 0, 1, 3).reshape(nb, H, EMB_D)


# speedup vs baseline: 5.6844x; 1.0121x over previous
import functools

import jax
import jax.numpy as jnp
from jax import lax
from jax.experimental import pallas as pl
from jax.experimental.pallas import tpu as pltpu
from jax.experimental.pallas import tpu_sc as plsc

EMB_D = 32
LANE = 128
NBT = 8
BLK = LANE * NBT
NDT = EMB_D // 8


def _sc_embed(table, idx4):
    nht, nrow, hsub, lane = idx4.shape
    H = nht * hsub
    assert lane == LANE and hsub == 8 and nrow % NBT == 0
    n_blk = nrow // NBT
    n_units = H * n_blk
    info = plsc.get_sparse_core_info()
    nc, ns = info.num_cores, info.num_subcores
    nw = nc * ns
    units_per_w = n_units // nw
    assert units_per_w * nw == n_units and units_per_w % 2 == 0

    mesh = plsc.VectorSubcoreMesh(core_axis_name="c", subcore_axis_name="s")

    @functools.partial(
        pl.kernel,
        mesh=mesh,
        out_type=jax.ShapeDtypeStruct((H, NDT, nrow * 8, LANE), jnp.float32),
        scratch_types=[
            pltpu.VMEM((NBT, 8, LANE), jnp.int32),
            pltpu.VMEM((NBT, 8, LANE), jnp.int32),
            pltpu.VMEM((BLK, EMB_D), jnp.float32),
            pltpu.VMEM((BLK, EMB_D), jnp.float32),
            pltpu.VMEM((NDT * NBT * 8, LANE), jnp.float32),
            pltpu.SemaphoreType.DMA,
            pltpu.SemaphoreType.DMA,
            pltpu.SemaphoreType.DMA,
        ],
        compiler_params=pltpu.CompilerParams(
            use_tc_tiling_on_sc=False, needs_layout_passes=False
        ),
    )
    def body(table_hbm, idx_hbm, out_hbm, idx_v0, idx_v1, rows_v0, rows_v1,
             trans_v, gs0, gs1, osem):
        wid = lax.axis_index("s") * nc + lax.axis_index("c")
        u0 = wid * units_per_w
        idx_vs = (idx_v0, idx_v1)
        rows_vs = (rows_v0, rows_v1)
        gsems = (gs0, gs1)
        iota = lax.iota(jnp.int32, 16)
        ms = [(iota + k) % 16 for k in range(16)]
        rowc = [(m // 8) * (NBT * 8) + (m % 8) for m in ms]

        def fire_unit(u, slot):
            h = u // n_blk
            ht, hr = h // 8, h % 8
            r = (u % n_blk) * NBT
            pltpu.sync_copy(idx_hbm.at[ht, pl.ds(r, NBT)], idx_vs[slot])
            for j in range(NBT):
                pltpu.async_copy(
                    table_hbm.at[idx_vs[slot].at[j, hr]],
                    rows_vs[slot].at[pl.ds(j * LANE, LANE)],
                    gsems[slot],
                )

        def wait_unit(slot):
            pltpu.make_async_copy(
                table_hbm.at[pl.ds(0, BLK)], rows_vs[slot], gsems[slot]
            ).wait()

        def transpose_block(slot):
            rows = rows_vs[slot]

            def bt_body(bt, c):
                base = bt * LANE
                for g16 in range(8):
                    toks = iota + (base + g16 * 16)
                    col = iota + (g16 * 16)
                    for d0 in (0, 16):
                        srow = (d0 // 8) * (NBT * 8) + bt * 8
                        vs = []
                        for k in range(16):
                            v = plsc.load_gather(rows, [toks, ms[k] + d0])
                            vs.append(v)
                        for k in range(16):
                            plsc.store_scatter(
                                trans_v, [rowc[k] + srow, col], vs[k]
                            )
                return c

            lax.fori_loop(0, NBT, bt_body, 0)

        def store_unit(u):
            h = u // n_blk
            r = (u % n_blk) * NBT * 8
            for dt in range(NDT):
                pltpu.async_copy(
                    trans_v.at[pl.ds(dt * NBT * 8, NBT * 8)],
                    out_hbm.at[h, dt, pl.ds(r, NBT * 8)],
                    osem,
                )

        def wait_store(u):
            h = u // n_blk
            r = (u % n_blk) * NBT * 8
            for dt in range(NDT):
                pltpu.make_async_copy(
                    trans_v.at[pl.ds(dt * NBT * 8, NBT * 8)],
                    out_hbm.at[h, dt, pl.ds(r, NBT * 8)],
                    osem,
                ).wait()

        fire_unit(u0, 0)

        def outer(g, carry):
            for b in range(2):
                i = g * 2 + b
                u = u0 + i
                slot = b
                wait_unit(slot)

                @pl.when(i + 1 < units_per_w)
                def _():
                    fire_unit(u + 1, 1 - slot)

                @pl.when(i >= 1)
                def _():
                    wait_store(u - 1)

                transpose_block(slot)
                store_unit(u)
            return carry

        lax.fori_loop(0, units_per_w // 2, outer, 0)
        wait_store(u0 + units_per_w - 1)

    return body(table, idx4)


def kernel(indices, table):
    nb, H = indices.shape
    assert nb % BLK == 0 and H % 8 == 0 and table.shape[1] == EMB_D
    idx4 = (
        indices.astype(jnp.int32)
        .reshape(nb // LANE, LANE, H // 8, 8)
        .transpose(2, 0, 3, 1)
    )
    x4 = _sc_embed(table, idx4)
    x5 = x4.reshape(H, NDT, nb // LANE, 8, LANE)
    return x5.transpose(2, 4, 0, 1, 3).reshape(nb, H, EMB_D)


# speedup vs baseline: 5.9626x; 1.0489x over previous
import functools

import jax
import jax.numpy as jnp
from jax import lax
from jax.experimental import pallas as pl
from jax.experimental.pallas import tpu as pltpu
from jax.experimental.pallas import tpu_sc as plsc

EMB_D = 32
LANE = 128
NBT = 8
BLK = LANE * NBT
NDT = EMB_D // 8


def _sc_embed(table, idx4):
    nht, nrow, hsub, lane = idx4.shape
    H = nht * hsub
    assert lane == LANE and hsub == 8 and nrow % NBT == 0
    n_blk = nrow // NBT
    n_units = H * n_blk
    info = plsc.get_sparse_core_info()
    nc, ns = info.num_cores, info.num_subcores
    nw = nc * ns
    units_per_w = n_units // nw
    assert units_per_w * nw == n_units and units_per_w % 2 == 0

    mesh = plsc.VectorSubcoreMesh(core_axis_name="c", subcore_axis_name="s")

    @functools.partial(
        pl.kernel,
        mesh=mesh,
        out_type=jax.ShapeDtypeStruct((H, NDT, nrow * 8, LANE), jnp.float32),
        scratch_types=[
            pltpu.VMEM((NBT, 8, LANE), jnp.int32),
            pltpu.VMEM((NBT, 8, LANE), jnp.int32),
            pltpu.VMEM((BLK, EMB_D), jnp.float32),
            pltpu.VMEM((BLK, EMB_D), jnp.float32),
            pltpu.VMEM((NDT * NBT * 8, LANE), jnp.float32),
            pltpu.SemaphoreType.DMA,
            pltpu.SemaphoreType.DMA,
            pltpu.SemaphoreType.DMA,
            pltpu.SemaphoreType.DMA,
            pltpu.SemaphoreType.DMA,
        ],
        compiler_params=pltpu.CompilerParams(
            use_tc_tiling_on_sc=False, needs_layout_passes=False
        ),
    )
    def body(table_hbm, idx_hbm, out_hbm, idx_v0, idx_v1, rows_v0, rows_v1,
             trans_v, gs0, gs1, is0, is1, osem):
        wid = lax.axis_index("s") * nc + lax.axis_index("c")
        u0 = wid * units_per_w
        idx_vs = (idx_v0, idx_v1)
        rows_vs = (rows_v0, rows_v1)
        gsems = (gs0, gs1)
        isems = (is0, is1)
        iota = lax.iota(jnp.int32, 16)
        ms = [(iota + k) % 16 for k in range(16)]
        rowc = [(m // 8) * (NBT * 8) + (m % 8) for m in ms]

        def idx_args(u, slot):
            h = u // n_blk
            r = (u % n_blk) * NBT
            return idx_hbm.at[h // 8, pl.ds(r, NBT)], idx_vs[slot], isems[slot]

        def fire_idx(u, slot):
            pltpu.async_copy(*idx_args(u, slot))

        def fire_gathers(u, slot):
            h = u // n_blk
            hr = h % 8
            pltpu.make_async_copy(*idx_args(u, slot)).wait()
            for j in range(NBT):
                pltpu.async_copy(
                    table_hbm.at[idx_vs[slot].at[j, hr]],
                    rows_vs[slot].at[pl.ds(j * LANE, LANE)],
                    gsems[slot],
                )

        def wait_unit(slot):
            pltpu.make_async_copy(
                table_hbm.at[pl.ds(0, BLK)], rows_vs[slot], gsems[slot]
            ).wait()

        def transpose_block(slot):
            rows = rows_vs[slot]

            def bt_body(bt, c):
                base = bt * LANE
                for g16 in range(8):
                    toks = iota + (base + g16 * 16)
                    col = iota + (g16 * 16)
                    for d0 in (0, 16):
                        srow = (d0 // 8) * (NBT * 8) + bt * 8
                        vs = []
                        for k in range(16):
                            v = plsc.load_gather(rows, [toks, ms[k] + d0])
                            vs.append(v)
                        for k in range(16):
                            plsc.store_scatter(
                                trans_v, [rowc[k] + srow, col], vs[k]
                            )
                return c

            lax.fori_loop(0, NBT, bt_body, 0)

        def store_unit(u):
            h = u // n_blk
            r = (u % n_blk) * NBT * 8
            for dt in range(NDT):
                pltpu.async_copy(
                    trans_v.at[pl.ds(dt * NBT * 8, NBT * 8)],
                    out_hbm.at[h, dt, pl.ds(r, NBT * 8)],
                    osem,
                )

        def wait_store(u):
            h = u // n_blk
            r = (u % n_blk) * NBT * 8
            for dt in range(NDT):
                pltpu.make_async_copy(
                    trans_v.at[pl.ds(dt * NBT * 8, NBT * 8)],
                    out_hbm.at[h, dt, pl.ds(r, NBT * 8)],
                    osem,
                ).wait()

        fire_idx(u0, 0)
        fire_gathers(u0, 0)
        fire_idx(u0 + 1, 1)

        def outer(g, carry):
            for b in range(2):
                i = g * 2 + b
                u = u0 + i
                slot = b
                wait_unit(slot)

                @pl.when(i + 1 < units_per_w)
                def _():
                    fire_gathers(u + 1, 1 - slot)

                @pl.when(i + 2 < units_per_w)
                def _():
                    fire_idx(u + 2, slot)

                @pl.when(i >= 1)
                def _():
                    wait_store(u - 1)

                transpose_block(slot)
                store_unit(u)
            return carry

        lax.fori_loop(0, units_per_w // 2, outer, 0)
        wait_store(u0 + units_per_w - 1)

    return body(table, idx4)


def kernel(indices, table):
    nb, H = indices.shape
    assert nb % BLK == 0 and H % 8 == 0 and table.shape[1] == EMB_D
    idx4 = (
        indices.astype(jnp.int32)
        .reshape(nb // LANE, LANE, H // 8, 8)
        .transpose(2, 0, 3, 1)
    )
    x4 = _sc_embed(table, idx4)
    x5 = x4.reshape(H, NDT, nb // LANE, 8, LANE)
    return x5.transpose(2, 4, 0, 1, 3).reshape(nb, H, EMB_D)
